# Initial kernel scaffold; baseline (speedup 1.0000x reference)
import jax
import jax.numpy as jnp
from jax.experimental import pallas as pl


def kernel(input_ids, word_emb, pos_emb, qw, kw, vw, ow, ln1_s, ln1_b, ln2_s, ln2_b, ff1_w, ff1_b, ff2_w, ff2_b, lnf_s, lnf_b, dense_w, w_ih, w_hh):
    raise NotImplementedError("write your pallas kernel here")


# speedup vs baseline: 2.2962x; 2.2962x over previous
import functools

import jax
import jax.numpy as jnp
from jax.experimental import pallas as pl
from jax.experimental.pallas import tpu as pltpu
from jax.experimental.pallas import tpu_sc as plsc

V = 32000
H = 768
FF = 3072
NH = 12
B = 2
S = 2048
CHUNK = 64
D = H // NH
WIN = 3 * CHUNK
NQ = 2 * CHUNK
NIDS = B * WIN
GW = 16
VT = 3200

_dot = functools.partial(jnp.dot, precision=jax.lax.Precision.HIGHEST,
                         preferred_element_type=jnp.float32)


def _dot_t(a, b):
    return jax.lax.dot_general(a, b, (((1,), (1,)), ((), ())),
                               precision=jax.lax.Precision.HIGHEST,
                               preferred_element_type=jnp.float32)


def _ln(x, s, b, eps=1e-12):
    m = jnp.mean(x, axis=-1, keepdims=True)
    v = jnp.mean((x - m) ** 2, axis=-1, keepdims=True)
    return (x - m) / jnp.sqrt(v + eps) * s + b


def _gather_rows(table, ids_2d):
    mesh = plsc.VectorSubcoreMesh(core_axis_name="core",
                                  subcore_axis_name="subcore")

    @pl.kernel(out_type=jax.ShapeDtypeStruct((NIDS, H), jnp.float32),
               mesh=mesh)
    def kern(x_hbm, i_hbm, o_hbm):
        def body(i_vmem, o_vmem):
            pltpu.sync_copy(x_hbm.at[i_vmem.at[0]], o_vmem)

        pltpu.emit_pipeline(
            body,
            grid=(NIDS // GW,),
            in_specs=[pl.BlockSpec((1, GW), index_map=lambda i: (i, 0))],
            out_specs=[pl.BlockSpec((GW, H), index_map=lambda i: (i, 0))],
            core_axis_name="subcore",
            dimension_semantics=(pltpu.PARALLEL,),
        )(i_hbm, o_hbm)

    return kern(table, ids_2d)


def _layer0_body(g_ref, pos_ref, qw_ref, kw_ref, vw_ref, ow_ref,
                 l1s_ref, l1b_ref, l2s_ref, l2b_ref,
                 f1w_ref, f1b_ref, f2w_ref, f2b_ref,
                 x1_ref, x2_ref):
    pos = pos_ref[...]
    g = g_ref[...]
    e_all = jnp.concatenate(
        [g[b * WIN:(b + 1) * WIN] + pos for b in range(B)], axis=0)
    xn = _ln(e_all, l1s_ref[...], l1b_ref[...])
    q_all = _dot(xn, qw_ref[...])
    k_all = _dot(xn, kw_ref[...])
    v_all = _dot(xn, vw_ref[...])

    qi = jax.lax.broadcasted_iota(jnp.int32, (NQ, WIN), 0)
    kj = jax.lax.broadcasted_iota(jnp.int32, (NQ, WIN), 1)
    lo = (qi // CHUNK) * CHUNK
    mask = (kj >= lo) & (kj < lo + 2 * CHUNK)

    x2_rows = []
    for b in range(B):
        q = q_all[b * WIN + CHUNK:(b + 1) * WIN]
        k = k_all[b * WIN:(b + 1) * WIN]
        v = v_all[b * WIN:(b + 1) * WIN]
        outs = []
        for h in range(NH):
            sc = _dot_t(q[:, h * D:(h + 1) * D], k[:, h * D:(h + 1) * D])
            sc = sc * (1.0 / 8.0)
            sc = jnp.where(mask, sc, -1e9)
            p = jax.nn.softmax(sc, axis=-1)
            outs.append(_dot(p, v[:, h * D:(h + 1) * D]))
        attn = jnp.concatenate(outs, axis=-1)
        e_b = e_all[b * WIN + CHUNK:(b + 1) * WIN]
        x2_rows.append(e_b + _dot(attn, ow_ref[...]))

    x2_all = jnp.concatenate(x2_rows, axis=0)
    h2 = _ln(x2_all, l2s_ref[...], l2b_ref[...])
    ffo = _dot(jax.nn.relu(_dot(h2, f1w_ref[...]) + f1b_ref[...]),
               f2w_ref[...]) + f2b_ref[...]
    for b in range(B):
        e_b = e_all[b * WIN + CHUNK:(b + 1) * WIN]
        x1_ref[b] = e_b + ffo[b * NQ:(b + 1) * NQ]
        x2_ref[b] = x2_all[b * NQ:(b + 1) * NQ]


def _layer0_call(g, pos, qw0, kw0, vw0, ow0, l1s, l1b, l2s, l2b,
                 f1w, f1b, f2w, f2b, interpret=False):
    out_shape = [jax.ShapeDtypeStruct((B, NQ, H), jnp.float32),
                 jax.ShapeDtypeStruct((B, NQ, H), jnp.float32)]
    return pl.pallas_call(
        _layer0_body, out_shape=out_shape, interpret=interpret,
    )(g, pos, qw0, kw0, vw0, ow0, l1s, l1b, l2s, l2b, f1w, f1b, f2w, f2b)


def _layer1_body(x1_ref, x2row_ref, el_ref, qw_ref, kw_ref, vw_ref, ow_ref,
                 l1s_ref, l1b_ref, l2s_ref, l2b_ref,
                 f1w_ref, f1b_ref, f2w_ref, f2b_ref,
                 lnfs_ref, lnfb_ref, dw_ref, wih_ref, whh_ref,
                 out_ref):
    P = 126
    x2row = x2row_ref[...]
    el = el_ref[...]
    for b in range(B):
        x1 = x1_ref[b]
        xn = _ln(x1, l1s_ref[...], l1b_ref[...])
        k = _dot(xn, kw_ref[...])
        v = _dot(xn, vw_ref[...])
        qrow = _dot(xn[P:P + 1], qw_ref[...])
        outs = []
        for h in range(NH):
            sc = _dot_t(qrow[:, h * D:(h + 1) * D],
                        k[:, h * D:(h + 1) * D]) * (1.0 / 8.0)
            p = jax.nn.softmax(sc, axis=-1)
            outs.append(_dot(p, v[:, h * D:(h + 1) * D]))
        attn = jnp.concatenate(outs, axis=-1)
        x2f = x2row[b:b + 1] + _dot(attn, ow_ref[...])
        h2 = _ln(x2f, l2s_ref[...], l2b_ref[...])
        ffo = _dot(jax.nn.relu(_dot(h2, f1w_ref[...]) + f1b_ref[...]),
                   f2w_ref[...]) + f2b_ref[...]
        x1f = x1[P:P + 1] + ffo
        hs = jnp.concatenate([x2f, x1f], axis=-1)
        hsn = _ln(hs, lnfs_ref[...], lnfb_ref[...])
        ht = _dot(hsn, dw_ref[...])
        gates = _dot(el[b:b + 1], wih_ref[...]) + _dot(ht, whh_ref[...])
        gi = gates[:, 0 * H:1 * H]
        gf = gates[:, 1 * H:2 * H]
        gg = gates[:, 2 * H:3 * H]
        go = gates[:, 3 * H:4 * H]
        c = jax.nn.sigmoid(gf) * ht + jax.nn.sigmoid(gi) * jnp.tanh(gg)
        hout = jax.nn.sigmoid(go) * jnp.tanh(c)
        out_ref[b, :] = hout[0]


def _layer1_call(x1, x2row, el, qw1, kw1, vw1, ow1, l1s, l1b, l2s, l2b,
                 f1w, f1b, f2w, f2b, lnfs, lnfb, dw, wih, whh,
                 interpret=False):
    return pl.pallas_call(
        _layer1_body,
        out_shape=jax.ShapeDtypeStruct((B, H), jnp.float32),
        interpret=interpret,
    )(x1, x2row, el, qw1, kw1, vw1, ow1, l1s, l1b, l2s, l2b,
      f1w, f1b, f2w, f2b, lnfs, lnfb, dw, wih, whh)


def _logits_body(h_ref, w_ref, o_ref):
    o_ref[...] = _dot_t(h_ref[...], w_ref[...])


def _logits_call(hout, word_emb, interpret=False):
    return pl.pallas_call(
        _logits_body,
        grid=(V // VT,),
        in_specs=[pl.BlockSpec((B, H), lambda j: (0, 0)),
                  pl.BlockSpec((VT, H), lambda j: (j, 0))],
        out_specs=pl.BlockSpec((B, VT), lambda j: (0, j)),
        out_shape=jax.ShapeDtypeStruct((B, V), jnp.float32),
        interpret=interpret,
    )(hout, word_emb)


def kernel(input_ids, word_emb, pos_emb, qw, kw, vw, ow, ln1_s, ln1_b,
           ln2_s, ln2_b, ff1_w, ff1_b, ff2_w, ff2_b, lnf_s, lnf_b,
           dense_w, w_ih, w_hh):
    ids = input_ids[:, S - WIN:]
    ids_2d = ids.reshape(NIDS // GW, GW).astype(jnp.int32)
    pos = pos_emb[S - WIN:S]

    g = _gather_rows(word_emb, ids_2d)
    emb_last = g[WIN - 1::WIN]

    x1, x2 = _layer0_call(
        g, pos, qw[0], kw[0], vw[0], ow[0], ln1_s[0], ln1_b[0],
        ln2_s[0], ln2_b[0], ff1_w[0], ff1_b[0], ff2_w[0], ff2_b[0])

    x2row = x2[:, 126, :]
    hout = _layer1_call(
        x1, x2row, emb_last, qw[1], kw[1], vw[1], ow[1], ln1_s[1], ln1_b[1],
        ln2_s[1], ln2_b[1], ff1_w[1], ff1_b[1], ff2_w[1], ff2_b[1],
        lnf_s, lnf_b, dense_w, w_ih, w_hh)

    return _logits_call(hout, word_emb)


# speedup vs baseline: 2.6250x; 1.1432x over previous
import functools

import jax
import jax.numpy as jnp
from jax.experimental import pallas as pl
from jax.experimental.pallas import tpu as pltpu
from jax.experimental.pallas import tpu_sc as plsc

V = 32000
H = 768
FF = 3072
NH = 12
B = 2
S = 2048
CHUNK = 64
D = H // NH
WIN = 3 * CHUNK
NQ = 2 * CHUNK
NIDS = B * WIN
GW = 16
VT = 3200

_dot = functools.partial(jnp.dot, precision=jax.lax.Precision.HIGHEST,
                         preferred_element_type=jnp.float32)


def _dot_t(a, b):
    return jax.lax.dot_general(a, b, (((1,), (1,)), ((), ())),
                               precision=jax.lax.Precision.HIGHEST,
                               preferred_element_type=jnp.float32)


def _ln(x, s, b, eps=1e-12):
    m = jnp.mean(x, axis=-1, keepdims=True)
    v = jnp.mean((x - m) ** 2, axis=-1, keepdims=True)
    return (x - m) / jnp.sqrt(v + eps) * s + b


def _gather_rows(table, ids_2d):
    mesh = plsc.VectorSubcoreMesh(core_axis_name="core",
                                  subcore_axis_name="subcore")

    @pl.kernel(out_type=jax.ShapeDtypeStruct((NIDS, H), jnp.float32),
               mesh=mesh)
    def kern(x_hbm, i_hbm, o_hbm):
        def body(i_vmem, o_vmem):
            pltpu.sync_copy(x_hbm.at[i_vmem.at[0]], o_vmem)

        pltpu.emit_pipeline(
            body,
            grid=(NIDS // GW,),
            in_specs=[pl.BlockSpec((1, GW), index_map=lambda i: (i, 0))],
            out_specs=[pl.BlockSpec((GW, H), index_map=lambda i: (i, 0))],
            core_axis_name="subcore",
            dimension_semantics=(pltpu.PARALLEL,),
        )(i_hbm, o_hbm)

    return kern(table, ids_2d)


def _layer0_body(g_ref, pos_ref, qw_ref, kw_ref, vw_ref, ow_ref,
                 l1s_ref, l1b_ref, l2s_ref, l2b_ref,
                 f1w_ref, f1b_ref, f2w_ref, f2b_ref,
                 x1_ref, x2_ref):
    pos = pos_ref[...]
    g = g_ref[...]
    e_all = jnp.concatenate(
        [g[b * WIN:(b + 1) * WIN] + pos for b in range(B)], axis=0)
    xn = _ln(e_all, l1s_ref[...], l1b_ref[...])
    q_all = _dot(xn, qw_ref[...])
    k_all = _dot(xn, kw_ref[...])
    v_all = _dot(xn, vw_ref[...])

    qi = jax.lax.broadcasted_iota(jnp.int32, (NQ, WIN), 0)
    kj = jax.lax.broadcasted_iota(jnp.int32, (NQ, WIN), 1)
    lo = (qi // CHUNK) * CHUNK
    mask = (kj >= lo) & (kj < lo + 2 * CHUNK)

    x2_rows = []
    for b in range(B):
        q = q_all[b * WIN + CHUNK:(b + 1) * WIN]
        k = k_all[b * WIN:(b + 1) * WIN]
        v = v_all[b * WIN:(b + 1) * WIN]
        outs = []
        for h in range(NH):
            sc = _dot_t(q[:, h * D:(h + 1) * D], k[:, h * D:(h + 1) * D])
            sc = sc * (1.0 / 8.0)
            sc = jnp.where(mask, sc, -1e9)
            p = jax.nn.softmax(sc, axis=-1)
            outs.append(_dot(p, v[:, h * D:(h + 1) * D]))
        attn = jnp.concatenate(outs, axis=-1)
        e_b = e_all[b * WIN + CHUNK:(b + 1) * WIN]
        x2_rows.append(e_b + _dot(attn, ow_ref[...]))

    x2_all = jnp.concatenate(x2_rows, axis=0)
    h2 = _ln(x2_all, l2s_ref[...], l2b_ref[...])
    ffo = _dot(jax.nn.relu(_dot(h2, f1w_ref[...]) + f1b_ref[...]),
               f2w_ref[...]) + f2b_ref[...]
    for b in range(B):
        e_b = e_all[b * WIN + CHUNK:(b + 1) * WIN]
        x1_ref[b] = e_b + ffo[b * NQ:(b + 1) * NQ]
        x2_ref[b] = x2_all[b * NQ:(b + 1) * NQ]


def _layer0_call(g, pos, qw0, kw0, vw0, ow0, l1s, l1b, l2s, l2b,
                 f1w, f1b, f2w, f2b, interpret=False):
    out_shape = [jax.ShapeDtypeStruct((B, NQ, H), jnp.float32),
                 jax.ShapeDtypeStruct((B, NQ, H), jnp.float32)]
    return pl.pallas_call(
        _layer0_body, out_shape=out_shape, interpret=interpret,
    )(g, pos, qw0, kw0, vw0, ow0, l1s, l1b, l2s, l2b, f1w, f1b, f2w, f2b)


def _layer1_body(x1_ref, x2row_ref, el_ref, qw_ref, kw_ref, vw_ref, ow_ref,
                 l1s_ref, l1b_ref, l2s_ref, l2b_ref,
                 f1w_ref, f1b_ref, f2w_ref, f2b_ref,
                 lnfs_ref, lnfb_ref, dw_ref, wih_ref, whh_ref,
                 out_ref):
    P = 126
    x2row = x2row_ref[...]
    el = el_ref[...]
    for b in range(B):
        x1 = x1_ref[b]
        xn = _ln(x1, l1s_ref[...], l1b_ref[...])
        k = _dot(xn, kw_ref[...])
        v = _dot(xn, vw_ref[...])
        qrow = _dot(xn[P:P + 1], qw_ref[...])
        outs = []
        for h in range(NH):
            sc = _dot_t(qrow[:, h * D:(h + 1) * D],
                        k[:, h * D:(h + 1) * D]) * (1.0 / 8.0)
            p = jax.nn.softmax(sc, axis=-1)
            outs.append(_dot(p, v[:, h * D:(h + 1) * D]))
        attn = jnp.concatenate(outs, axis=-1)
        x2f = x2row[b:b + 1] + _dot(attn, ow_ref[...])
        h2 = _ln(x2f, l2s_ref[...], l2b_ref[...])
        ffo = _dot(jax.nn.relu(_dot(h2, f1w_ref[...]) + f1b_ref[...]),
                   f2w_ref[...]) + f2b_ref[...]
        x1f = x1[P:P + 1] + ffo
        hs = jnp.concatenate([x2f, x1f], axis=-1)
        hsn = _ln(hs, lnfs_ref[...], lnfb_ref[...])
        ht = _dot(hsn, dw_ref[...])
        gates = _dot(el[b:b + 1], wih_ref[...]) + _dot(ht, whh_ref[...])
        gi = gates[:, 0 * H:1 * H]
        gf = gates[:, 1 * H:2 * H]
        gg = gates[:, 2 * H:3 * H]
        go = gates[:, 3 * H:4 * H]
        c = jax.nn.sigmoid(gf) * ht + jax.nn.sigmoid(gi) * jnp.tanh(gg)
        hout = jax.nn.sigmoid(go) * jnp.tanh(c)
        out_ref[b, :] = hout[0]


def _layer1_call(x1, x2row, el, qw1, kw1, vw1, ow1, l1s, l1b, l2s, l2b,
                 f1w, f1b, f2w, f2b, lnfs, lnfb, dw, wih, whh,
                 interpret=False):
    return pl.pallas_call(
        _layer1_body,
        out_shape=jax.ShapeDtypeStruct((B, H), jnp.float32),
        interpret=interpret,
    )(x1, x2row, el, qw1, kw1, vw1, ow1, l1s, l1b, l2s, l2b,
      f1w, f1b, f2w, f2b, lnfs, lnfb, dw, wih, whh)


def _logits_body(h_ref, w_ref, o_ref):
    w = w_ref[...]
    hv = h_ref[...]
    for b in range(B):
        hb = hv[b:b + 1]
        o_ref[:, b:b + 1] = jnp.sum(w * hb, axis=1, keepdims=True)


def _logits_call(hout, word_emb, interpret=False):
    out_t = pl.pallas_call(
        _logits_body,
        grid=(V // VT,),
        in_specs=[pl.BlockSpec((B, H), lambda j: (0, 0)),
                  pl.BlockSpec((VT, H), lambda j: (j, 0))],
        out_specs=pl.BlockSpec((VT, B), lambda j: (j, 0)),
        out_shape=jax.ShapeDtypeStruct((V, B), jnp.float32),
        interpret=interpret,
    )(hout, word_emb)
    return out_t.T


def kernel(input_ids, word_emb, pos_emb, qw, kw, vw, ow, ln1_s, ln1_b,
           ln2_s, ln2_b, ff1_w, ff1_b, ff2_w, ff2_b, lnf_s, lnf_b,
           dense_w, w_ih, w_hh):
    ids = input_ids[:, S - WIN:]
    ids_2d = ids.reshape(NIDS // GW, GW).astype(jnp.int32)
    pos = pos_emb[S - WIN:S]

    g = _gather_rows(word_emb, ids_2d)
    emb_last = g[WIN - 1::WIN]

    x1, x2 = _layer0_call(
        g, pos, qw[0], kw[0], vw[0], ow[0], ln1_s[0], ln1_b[0],
        ln2_s[0], ln2_b[0], ff1_w[0], ff1_b[0], ff2_w[0], ff2_b[0])

    x2row = x2[:, 126, :]
    hout = _layer1_call(
        x1, x2row, emb_last, qw[1], kw[1], vw[1], ow[1], ln1_s[1], ln1_b[1],
        ln2_s[1], ln2_b[1], ff1_w[1], ff1_b[1], ff2_w[1], ff2_b[1],
        lnf_s, lnf_b, dense_w, w_ih, w_hh)

    return _logits_call(hout, word_emb)


# speedup vs baseline: 3.2756x; 1.2478x over previous
import functools

import jax
import jax.numpy as jnp
from jax.experimental import pallas as pl
from jax.experimental.pallas import tpu as pltpu
from jax.experimental.pallas import tpu_sc as plsc

V = 32000
H = 768
FF = 3072
NH = 12
B = 2
S = 2048
CHUNK = 64
D = H // NH
WIN = 3 * CHUNK
NQ = 2 * CHUNK
NIDS = B * WIN
GW = 16
VT = 3200

_dot = functools.partial(jnp.dot, precision=jax.lax.Precision.HIGHEST,
                         preferred_element_type=jnp.float32)


def _dot_t(a, b):
    return jax.lax.dot_general(a, b, (((1,), (1,)), ((), ())),
                               precision=jax.lax.Precision.HIGHEST,
                               preferred_element_type=jnp.float32)


def _row_mm(x_row, w):
    return jnp.sum(x_row.reshape(-1, 1) * w, axis=0, keepdims=True)


def _ln(x, s, b, eps=1e-12):
    m = jnp.mean(x, axis=-1, keepdims=True)
    v = jnp.mean((x - m) ** 2, axis=-1, keepdims=True)
    return (x - m) / jnp.sqrt(v + eps) * s + b


def _gather_rows(table, ids_2d):
    mesh = plsc.VectorSubcoreMesh(core_axis_name="core",
                                  subcore_axis_name="subcore")

    @pl.kernel(out_type=jax.ShapeDtypeStruct((NIDS, H), jnp.float32),
               mesh=mesh)
    def kern(x_hbm, i_hbm, o_hbm):
        def body(i_vmem, o_vmem):
            pltpu.sync_copy(x_hbm.at[i_vmem.at[0]], o_vmem)

        pltpu.emit_pipeline(
            body,
            grid=(NIDS // GW,),
            in_specs=[pl.BlockSpec((1, GW), index_map=lambda i: (i, 0))],
            out_specs=[pl.BlockSpec((GW, H), index_map=lambda i: (i, 0))],
            core_axis_name="subcore",
            dimension_semantics=(pltpu.PARALLEL,),
        )(i_hbm, o_hbm)

    return kern(table, ids_2d)


def _layer0_body(g_ref, pos_ref, qw_ref, kw_ref, vw_ref, ow_ref,
                 l1s_ref, l1b_ref, l2s_ref, l2b_ref,
                 f1w_ref, f1b_ref, f2w_ref, f2b_ref,
                 x1_ref, x2_ref):
    pos = pos_ref[...]
    g = g_ref[...]
    e_all = jnp.concatenate(
        [g[b * WIN:(b + 1) * WIN] + pos for b in range(B)], axis=0)
    xn = _ln(e_all, l1s_ref[...], l1b_ref[...])
    q_in = jnp.concatenate(
        [xn[b * WIN + CHUNK:(b + 1) * WIN] for b in range(B)], axis=0)
    q_all = _dot(q_in, qw_ref[...])
    k_all = _dot(xn, kw_ref[...])
    v_all = _dot(xn, vw_ref[...])

    qi = jax.lax.broadcasted_iota(jnp.int32, (NQ, WIN), 0)
    kj = jax.lax.broadcasted_iota(jnp.int32, (NQ, WIN), 1)
    lo = (qi // CHUNK) * CHUNK
    mask = (kj >= lo) & (kj < lo + 2 * CHUNK)

    x2_rows = []
    for b in range(B):
        q = q_all[b * NQ:(b + 1) * NQ]
        k = k_all[b * WIN:(b + 1) * WIN]
        v = v_all[b * WIN:(b + 1) * WIN]
        outs = []
        for h in range(NH):
            sc = _dot_t(q[:, h * D:(h + 1) * D], k[:, h * D:(h + 1) * D])
            sc = sc * (1.0 / 8.0)
            sc = jnp.where(mask, sc, -1e9)
            p = jax.nn.softmax(sc, axis=-1)
            outs.append(_dot(p, v[:, h * D:(h + 1) * D]))
        attn = jnp.concatenate(outs, axis=-1)
        e_b = e_all[b * WIN + CHUNK:(b + 1) * WIN]
        x2_rows.append(e_b + _dot(attn, ow_ref[...]))

    x2_all = jnp.concatenate(x2_rows, axis=0)
    h2 = _ln(x2_all, l2s_ref[...], l2b_ref[...])
    ffo = _dot(jax.nn.relu(_dot(h2, f1w_ref[...]) + f1b_ref[...]),
               f2w_ref[...]) + f2b_ref[...]
    for b in range(B):
        e_b = e_all[b * WIN + CHUNK:(b + 1) * WIN]
        x1_ref[b] = e_b + ffo[b * NQ:(b + 1) * NQ]
        x2_ref[b] = x2_all[b * NQ:(b + 1) * NQ]


def _layer0_call(g, pos, qw0, kw0, vw0, ow0, l1s, l1b, l2s, l2b,
                 f1w, f1b, f2w, f2b, interpret=False):
    out_shape = [jax.ShapeDtypeStruct((B, NQ, H), jnp.float32),
                 jax.ShapeDtypeStruct((B, NQ, H), jnp.float32)]
    return pl.pallas_call(
        _layer0_body, out_shape=out_shape, interpret=interpret,
    )(g, pos, qw0, kw0, vw0, ow0, l1s, l1b, l2s, l2b, f1w, f1b, f2w, f2b)


def _layer1_body(x1_ref, x2row_ref, el_ref, qw_ref, kw_ref, vw_ref, ow_ref,
                 l1s_ref, l1b_ref, l2s_ref, l2b_ref,
                 f1w_ref, f1b_ref, f2w_ref, f2b_ref,
                 lnfs_ref, lnfb_ref, dw_ref, wih_ref, whh_ref,
                 out_ref):
    P = 126
    x2row = x2row_ref[...]
    el = el_ref[...]
    for b in range(B):
        x1 = x1_ref[b]
        xn = _ln(x1, l1s_ref[...], l1b_ref[...])
        k = _dot(xn, kw_ref[...])
        v = _dot(xn, vw_ref[...])
        qrow = _row_mm(xn[P:P + 1], qw_ref[...])
        outs = []
        for h in range(NH):
            kh = k[:, h * D:(h + 1) * D]
            vh = v[:, h * D:(h + 1) * D]
            qh = qrow[:, h * D:(h + 1) * D]
            sc = jnp.sum(kh * qh, axis=1, keepdims=True) * (1.0 / 8.0)
            sc = sc - jnp.max(sc, axis=0, keepdims=True)
            w = jnp.exp(sc)
            p = w / jnp.sum(w, axis=0, keepdims=True)
            outs.append(jnp.sum(p * vh, axis=0, keepdims=True))
        attn = jnp.concatenate(outs, axis=-1)
        x2f = x2row[b:b + 1] + _row_mm(attn, ow_ref[...])
        h2 = _ln(x2f, l2s_ref[...], l2b_ref[...])
        ffo = _row_mm(jax.nn.relu(_row_mm(h2, f1w_ref[...]) + f1b_ref[...]),
                      f2w_ref[...]) + f2b_ref[...]
        x1f = x1[P:P + 1] + ffo
        hs = jnp.concatenate([x2f, x1f], axis=-1)
        hsn = _ln(hs, lnfs_ref[...], lnfb_ref[...])
        ht = _row_mm(hsn, dw_ref[...])
        gates = (_row_mm(el[b:b + 1], wih_ref[...]) +
                 _row_mm(ht, whh_ref[...]))
        gi = gates[:, 0 * H:1 * H]
        gf = gates[:, 1 * H:2 * H]
        gg = gates[:, 2 * H:3 * H]
        go = gates[:, 3 * H:4 * H]
        c = jax.nn.sigmoid(gf) * ht + jax.nn.sigmoid(gi) * jnp.tanh(gg)
        hout = jax.nn.sigmoid(go) * jnp.tanh(c)
        out_ref[b, :] = hout[0]


def _layer1_call(x1, x2row, el, qw1, kw1, vw1, ow1, l1s, l1b, l2s, l2b,
                 f1w, f1b, f2w, f2b, lnfs, lnfb, dw, wih, whh,
                 interpret=False):
    return pl.pallas_call(
        _layer1_body,
        out_shape=jax.ShapeDtypeStruct((B, H), jnp.float32),
        interpret=interpret,
    )(x1, x2row, el, qw1, kw1, vw1, ow1, l1s, l1b, l2s, l2b,
      f1w, f1b, f2w, f2b, lnfs, lnfb, dw, wih, whh)


def _logits_body(h_ref, w_ref, o_ref):
    w = w_ref[...]
    hv = h_ref[...]
    for b in range(B):
        hb = hv[b:b + 1]
        o_ref[:, b:b + 1] = jnp.sum(w * hb, axis=1, keepdims=True)


def _logits_call(hout, word_emb, interpret=False):
    out_t = pl.pallas_call(
        _logits_body,
        grid=(V // VT,),
        in_specs=[pl.BlockSpec((B, H), lambda j: (0, 0)),
                  pl.BlockSpec((VT, H), lambda j: (j, 0))],
        out_specs=pl.BlockSpec((VT, B), lambda j: (j, 0)),
        out_shape=jax.ShapeDtypeStruct((V, B), jnp.float32),
        interpret=interpret,
    )(hout, word_emb)
    return out_t.T


def kernel(input_ids, word_emb, pos_emb, qw, kw, vw, ow, ln1_s, ln1_b,
           ln2_s, ln2_b, ff1_w, ff1_b, ff2_w, ff2_b, lnf_s, lnf_b,
           dense_w, w_ih, w_hh):
    ids = input_ids[:, S - WIN:]
    ids_2d = ids.reshape(NIDS // GW, GW).astype(jnp.int32)
    pos = pos_emb[S - WIN:S]

    g = _gather_rows(word_emb, ids_2d)
    emb_last = g[WIN - 1::WIN]

    x1, x2 = _layer0_call(
        g, pos, qw[0], kw[0], vw[0], ow[0], ln1_s[0], ln1_b[0],
        ln2_s[0], ln2_b[0], ff1_w[0], ff1_b[0], ff2_w[0], ff2_b[0])

    x2row = x2[:, 126, :]
    hout = _layer1_call(
        x1, x2row, emb_last, qw[1], kw[1], vw[1], ow[1], ln1_s[1], ln1_b[1],
        ln2_s[1], ln2_b[1], ff1_w[1], ff1_b[1], ff2_w[1], ff2_b[1],
        lnf_s, lnf_b, dense_w, w_ih, w_hh)

    return _logits_call(hout, word_emb)


# speedup vs baseline: 4.3916x; 1.3407x over previous
import functools

import jax
import jax.numpy as jnp
from jax.experimental import pallas as pl
from jax.experimental.pallas import tpu as pltpu
from jax.experimental.pallas import tpu_sc as plsc

V = 32000
H = 768
FF = 3072
NH = 12
B = 2
S = 2048
CHUNK = 64
D = H // NH
WIN = 3 * CHUNK
NQ = 2 * CHUNK
NIDS = B * WIN
GW = 24
VT = 640
NT = V // VT
P = 126

_dot = functools.partial(jnp.dot, precision=jax.lax.Precision.HIGHEST,
                         preferred_element_type=jnp.float32)


def _dot_t(a, b):
    return jax.lax.dot_general(a, b, (((1,), (1,)), ((), ())),
                               precision=jax.lax.Precision.HIGHEST,
                               preferred_element_type=jnp.float32)


def _row_mm(x_row, w):
    return jnp.sum(x_row.reshape(-1, 1) * w, axis=0, keepdims=True)


def _ln(x, s, b, eps=1e-12):
    m = jnp.mean(x, axis=-1, keepdims=True)
    v = jnp.mean((x - m) ** 2, axis=-1, keepdims=True)
    return (x - m) / jnp.sqrt(v + eps) * s + b


def _gather_rows(table, ids_2d):
    mesh = plsc.VectorSubcoreMesh(core_axis_name="core",
                                  subcore_axis_name="subcore")

    @pl.kernel(out_type=jax.ShapeDtypeStruct((NIDS, H), jnp.float32),
               mesh=mesh)
    def kern(x_hbm, i_hbm, o_hbm):
        def body(i_vmem, o_vmem):
            pltpu.sync_copy(x_hbm.at[i_vmem.at[0]], o_vmem)

        pltpu.emit_pipeline(
            body,
            grid=(NIDS // GW,),
            in_specs=[pl.BlockSpec((1, GW), index_map=lambda i: (i, 0))],
            out_specs=[pl.BlockSpec((GW, H), index_map=lambda i: (i, 0))],
            core_axis_name="subcore",
            dimension_semantics=(pltpu.PARALLEL,),
        )(i_hbm, o_hbm)

    return kern(table, ids_2d)


def _fused_body(g_ref, pos_ref, l1s_ref, l1b_ref, l2s_ref, l2b_ref,
                f1b_ref, f2b_ref, lnfs_ref, lnfb_ref,
                qw_ref, kw_ref, vw_ref, ow_ref, f1w_ref, f2w_ref,
                dw_ref, wih_ref, whh_ref, we_ref,
                o_ref,
                a0, a1, f1buf, f2buf, vb, sem, vsem):
    cps = {
        "q0": pltpu.make_async_copy(qw_ref.at[0], a0.at[0], sem.at[0]),
        "k0": pltpu.make_async_copy(kw_ref.at[0], a0.at[1], sem.at[1]),
        "v0": pltpu.make_async_copy(vw_ref.at[0], a0.at[2], sem.at[2]),
        "o0": pltpu.make_async_copy(ow_ref.at[0], a0.at[3], sem.at[3]),
        "f10": pltpu.make_async_copy(f1w_ref.at[0], f1buf, sem.at[4]),
        "f20": pltpu.make_async_copy(f2w_ref.at[0], f2buf, sem.at[5]),
        "q1": pltpu.make_async_copy(qw_ref.at[1], a1.at[0], sem.at[6]),
        "k1": pltpu.make_async_copy(kw_ref.at[1], a1.at[1], sem.at[7]),
        "v1": pltpu.make_async_copy(vw_ref.at[1], a1.at[2], sem.at[8]),
        "o1": pltpu.make_async_copy(ow_ref.at[1], a1.at[3], sem.at[9]),
    }
    for c in cps.values():
        c.start()

    def vocab_cp(i):
        return pltpu.make_async_copy(we_ref.at[pl.ds(i * VT, VT)],
                                     vb.at[i % 2], vsem.at[i % 2])

    vcp0 = vocab_cp(0)
    vcp1 = vocab_cp(1)
    vcp0.start()
    vcp1.start()

    pos = pos_ref[...]
    g = g_ref[...]
    e_all = jnp.concatenate(
        [g[b * WIN:(b + 1) * WIN] + pos for b in range(B)], axis=0)
    xn = _ln(e_all, l1s_ref[0], l1b_ref[0])
    q_in = jnp.concatenate(
        [xn[b * WIN + CHUNK:(b + 1) * WIN] for b in range(B)], axis=0)

    cps["q0"].wait()
    q_all = _dot(q_in, a0[0])
    cps["k0"].wait()
    k_all = _dot(xn, a0[1])
    cp_dw = pltpu.make_async_copy(dw_ref, a0.at[pl.ds(0, 2)], sem.at[12])
    cp_dw.start()
    cps["v0"].wait()
    v_all = _dot(xn, a0[2])

    qi = jax.lax.broadcasted_iota(jnp.int32, (NQ, WIN), 0)
    kj = jax.lax.broadcasted_iota(jnp.int32, (NQ, WIN), 1)
    lo = (qi // CHUNK) * CHUNK
    mask = (kj >= lo) & (kj < lo + 2 * CHUNK)

    cps["o0"].wait()
    x2_rows = []
    for b in range(B):
        q = q_all[b * NQ:(b + 1) * NQ]
        k = k_all[b * WIN:(b + 1) * WIN]
        v = v_all[b * WIN:(b + 1) * WIN]
        outs = []
        for h in range(NH):
            sc = _dot_t(q[:, h * D:(h + 1) * D], k[:, h * D:(h + 1) * D])
            sc = sc * (1.0 / 8.0)
            sc = jnp.where(mask, sc, -1e9)
            p = jax.nn.softmax(sc, axis=-1)
            outs.append(_dot(p, v[:, h * D:(h + 1) * D]))
        attn = jnp.concatenate(outs, axis=-1)
        e_b = e_all[b * WIN + CHUNK:(b + 1) * WIN]
        x2_rows.append(e_b + _dot(attn, a0[3]))

    x2_all = jnp.concatenate(x2_rows, axis=0)
    h2 = _ln(x2_all, l2s_ref[0], l2b_ref[0])
    cps["f10"].wait()
    h1 = jax.nn.relu(_dot(h2, f1buf[...]) + f1b_ref[0])
    cps["f20"].wait()
    ffo = _dot(h1, f2buf[...]) + f2b_ref[0]
    x1_all = jnp.concatenate(
        [e_all[b * WIN + CHUNK:(b + 1) * WIN] for b in range(B)],
        axis=0) + ffo

    cp_f11 = pltpu.make_async_copy(f1w_ref.at[1], f1buf, sem.at[10])
    cp_f21 = pltpu.make_async_copy(f2w_ref.at[1], f2buf, sem.at[11])
    cp_f11.start()
    cp_f21.start()

    xn1 = _ln(x1_all, l1s_ref[1], l1b_ref[1])
    cps["k1"].wait()
    k1_all = _dot(xn1, a1[1])
    cps["v1"].wait()
    v1_all = _dot(xn1, a1[2])
    cps["q1"].wait()
    cps["o1"].wait()
    cp_f11.wait()
    cp_f21.wait()
    hsn_rows = []
    for b in range(B):
        xb = xn1[b * NQ:(b + 1) * NQ]
        qrow = _row_mm(xb[P:P + 1], a1[0])
        k = k1_all[b * NQ:(b + 1) * NQ]
        v = v1_all[b * NQ:(b + 1) * NQ]
        outs = []
        for h in range(NH):
            kh = k[:, h * D:(h + 1) * D]
            vh = v[:, h * D:(h + 1) * D]
            qh = qrow[:, h * D:(h + 1) * D]
            sc = jnp.sum(kh * qh, axis=1, keepdims=True) * (1.0 / 8.0)
            sc = sc - jnp.max(sc, axis=0, keepdims=True)
            w = jnp.exp(sc)
            pb = w / jnp.sum(w, axis=0, keepdims=True)
            outs.append(jnp.sum(pb * vh, axis=0, keepdims=True))
        attn = jnp.concatenate(outs, axis=-1)
        x2f = x2_all[b * NQ + P:b * NQ + P + 1] + _row_mm(attn, a1[3])
        h2f = _ln(x2f, l2s_ref[1], l2b_ref[1])
        ffo1 = _row_mm(jax.nn.relu(_row_mm(h2f, f1buf[...]) + f1b_ref[1]),
                       f2buf[...]) + f2b_ref[1]
        x1f = x1_all[b * NQ + P:b * NQ + P + 1] + ffo1
        hs = jnp.concatenate([x2f, x1f], axis=-1)
        hsn_rows.append(_ln(hs, lnfs_ref[...], lnfb_ref[...]))

    cp_ih = pltpu.make_async_copy(wih_ref, f1buf, sem.at[13])
    cp_ih.start()
    cp_dw.wait()
    ht_rows = []
    el_rows = []
    for b in range(B):
        hsn_b = hsn_rows[b]
        ht_rows.append(_row_mm(hsn_b[:, :H], a0[0]) +
                       _row_mm(hsn_b[:, H:], a0[1]))
        el_rows.append(g_ref[(b + 1) * WIN - 1:(b + 1) * WIN])
    cp_ih.wait()
    gates_ih = [_row_mm(el_rows[b], f1buf[...]) for b in range(B)]
    cp_hh = pltpu.make_async_copy(whh_ref, f1buf, sem.at[14])
    cp_hh.start()
    cp_hh.wait()
    hout = []
    for b in range(B):
        ht = ht_rows[b]
        gates = gates_ih[b] + _row_mm(ht, f1buf[...])
        gi = gates[:, 0 * H:1 * H]
        gf = gates[:, 1 * H:2 * H]
        gg = gates[:, 2 * H:3 * H]
        go = gates[:, 3 * H:4 * H]
        c = jax.nn.sigmoid(gf) * ht + jax.nn.sigmoid(gi) * jnp.tanh(gg)
        hout.append(jax.nn.sigmoid(go) * jnp.tanh(c))

    for i in range(NT):
        vocab_cp(i).wait()
        w = vb[i % 2]
        for b in range(B):
            col = jnp.sum(w * hout[b], axis=1, keepdims=True)
            o_ref[b:b + 1, i * VT:(i + 1) * VT] = jnp.swapaxes(col, 0, 1)
        if i + 2 < NT:
            vocab_cp(i + 2).start()


def _fused_call(g, pos, ln1_s, ln1_b, ln2_s, ln2_b, ff1_b, ff2_b,
                lnf_s, lnf_b, qw, kw, vw, ow, ff1_w, ff2_w,
                dense_w, w_ih, w_hh, word_emb):
    n_vmem = 10
    n_any = 10
    in_specs = ([pl.BlockSpec(memory_space=pltpu.VMEM)] * n_vmem +
                [pl.BlockSpec(memory_space=pl.ANY)] * n_any)
    return pl.pallas_call(
        _fused_body,
        in_specs=in_specs,
        out_specs=pl.BlockSpec(memory_space=pltpu.VMEM),
        out_shape=jax.ShapeDtypeStruct((B, V), jnp.float32),
        scratch_shapes=[
            pltpu.VMEM((4, H, H), jnp.float32),
            pltpu.VMEM((4, H, H), jnp.float32),
            pltpu.VMEM((H, FF), jnp.float32),
            pltpu.VMEM((FF, H), jnp.float32),
            pltpu.VMEM((2, VT, H), jnp.float32),
            pltpu.SemaphoreType.DMA((15,)),
            pltpu.SemaphoreType.DMA((2,)),
        ],
    )(g, pos, ln1_s, ln1_b, ln2_s, ln2_b, ff1_b, ff2_b, lnf_s, lnf_b,
      qw, kw, vw, ow, ff1_w, ff2_w,
      dense_w.reshape(2, H, H), w_ih, w_hh, word_emb)


def kernel(input_ids, word_emb, pos_emb, qw, kw, vw, ow, ln1_s, ln1_b,
           ln2_s, ln2_b, ff1_w, ff1_b, ff2_w, ff2_b, lnf_s, lnf_b,
           dense_w, w_ih, w_hh):
    ids = input_ids[:, S - WIN:]
    ids_2d = ids.reshape(NIDS // GW, GW).astype(jnp.int32)
    pos = pos_emb[S - WIN:S]

    g = _gather_rows(word_emb, ids_2d)
    return _fused_call(g, pos, ln1_s, ln1_b, ln2_s, ln2_b, ff1_b, ff2_b,
                       lnf_s, lnf_b, qw, kw, vw, ow, ff1_w, ff2_w,
                       dense_w, w_ih, w_hh, word_emb)


# speedup vs baseline: 4.8210x; 1.0978x over previous
import functools

import jax
import jax.numpy as jnp
from jax.experimental import pallas as pl
from jax.experimental.pallas import tpu as pltpu
from jax.experimental.pallas import tpu_sc as plsc

V = 32000
H = 768
FF = 3072
NH = 12
B = 2
S = 2048
CHUNK = 64
D = H // NH
WIN = 3 * CHUNK
NQ = 2 * CHUNK
NIDS = B * WIN
GW = 64
VT = 3200
P = 126

_dot = functools.partial(jnp.dot, precision=jax.lax.Precision.HIGHEST,
                         preferred_element_type=jnp.float32)


def _dot_t(a, b):
    return jax.lax.dot_general(a, b, (((1,), (1,)), ((), ())),
                               precision=jax.lax.Precision.HIGHEST,
                               preferred_element_type=jnp.float32)


def _row_mm(x_row, w):
    return jnp.sum(x_row.reshape(-1, 1) * w, axis=0, keepdims=True)


def _ln(x, s, b, eps=1e-12):
    m = jnp.mean(x, axis=-1, keepdims=True)
    v = jnp.mean((x - m) ** 2, axis=-1, keepdims=True)
    return (x - m) / jnp.sqrt(v + eps) * s + b


def _gather_rows(table, ids_2d):
    mesh = plsc.VectorSubcoreMesh(core_axis_name="core",
                                  subcore_axis_name="subcore")

    @pl.kernel(out_type=jax.ShapeDtypeStruct((NIDS, H), jnp.float32),
               mesh=mesh)
    def kern(x_hbm, i_hbm, o_hbm):
        def body(i_vmem, o_vmem):
            pltpu.sync_copy(x_hbm.at[i_vmem.at[0]], o_vmem)

        pltpu.emit_pipeline(
            body,
            grid=(NIDS // GW,),
            in_specs=[pl.BlockSpec((1, GW), index_map=lambda i: (i, 0))],
            out_specs=[pl.BlockSpec((GW, H), index_map=lambda i: (i, 0))],
            core_axis_name="subcore",
            dimension_semantics=(pltpu.PARALLEL,),
        )(i_hbm, o_hbm)

    return kern(table, ids_2d)


def _cone_body(g_ref, pos_ref, l1s_ref, l1b_ref, l2s_ref, l2b_ref,
               f1b_ref, f2b_ref, lnfs_ref, lnfb_ref,
               qw_ref, kw_ref, vw_ref, ow_ref, f1w_ref, f2w_ref,
               hsn_ref,
               a0, a1, f1buf, f2buf, sem):
    cps = {
        "q0": pltpu.make_async_copy(qw_ref.at[0], a0.at[0], sem.at[0]),
        "k0": pltpu.make_async_copy(kw_ref.at[0], a0.at[1], sem.at[1]),
        "v0": pltpu.make_async_copy(vw_ref.at[0], a0.at[2], sem.at[2]),
        "o0": pltpu.make_async_copy(ow_ref.at[0], a0.at[3], sem.at[3]),
        "f10": pltpu.make_async_copy(f1w_ref.at[0], f1buf, sem.at[4]),
        "f20": pltpu.make_async_copy(f2w_ref.at[0], f2buf, sem.at[5]),
        "q1": pltpu.make_async_copy(qw_ref.at[1], a1.at[0], sem.at[6]),
        "k1": pltpu.make_async_copy(kw_ref.at[1], a1.at[1], sem.at[7]),
        "v1": pltpu.make_async_copy(vw_ref.at[1], a1.at[2], sem.at[8]),
        "o1": pltpu.make_async_copy(ow_ref.at[1], a1.at[3], sem.at[9]),
    }
    for c in cps.values():
        c.start()

    pos = pos_ref[...]
    g = g_ref[...]
    e_all = jnp.concatenate(
        [g[b * WIN:(b + 1) * WIN] + pos for b in range(B)], axis=0)
    xn = _ln(e_all, l1s_ref[0], l1b_ref[0])
    q_in = jnp.concatenate(
        [xn[b * WIN + CHUNK:(b + 1) * WIN] for b in range(B)], axis=0)

    cps["q0"].wait()
    q_all = _dot(q_in, a0[0])
    cps["k0"].wait()
    k_all = _dot(xn, a0[1])
    cps["v0"].wait()
    v_all = _dot(xn, a0[2])

    qi = jax.lax.broadcasted_iota(jnp.int32, (NQ, WIN), 0)
    kj = jax.lax.broadcasted_iota(jnp.int32, (NQ, WIN), 1)
    lo = (qi // CHUNK) * CHUNK
    mask = (kj >= lo) & (kj < lo + 2 * CHUNK)

    cps["o0"].wait()
    x2_rows = []
    for b in range(B):
        q = q_all[b * NQ:(b + 1) * NQ]
        k = k_all[b * WIN:(b + 1) * WIN]
        v = v_all[b * WIN:(b + 1) * WIN]
        outs = []
        for h in range(NH):
            sc = _dot_t(q[:, h * D:(h + 1) * D], k[:, h * D:(h + 1) * D])
            sc = sc * (1.0 / 8.0)
            sc = jnp.where(mask, sc, -1e9)
            p = jax.nn.softmax(sc, axis=-1)
            outs.append(_dot(p, v[:, h * D:(h + 1) * D]))
        attn = jnp.concatenate(outs, axis=-1)
        e_b = e_all[b * WIN + CHUNK:(b + 1) * WIN]
        x2_rows.append(e_b + _dot(attn, a0[3]))

    x2_all = jnp.concatenate(x2_rows, axis=0)
    h2 = _ln(x2_all, l2s_ref[0], l2b_ref[0])
    cps["f10"].wait()
    h1 = jax.nn.relu(_dot(h2, f1buf[...]) + f1b_ref[0])
    cps["f20"].wait()
    ffo = _dot(h1, f2buf[...]) + f2b_ref[0]
    x1_all = jnp.concatenate(
        [e_all[b * WIN + CHUNK:(b + 1) * WIN] for b in range(B)],
        axis=0) + ffo

    cp_f11 = pltpu.make_async_copy(f1w_ref.at[1], f1buf, sem.at[10])
    cp_f21 = pltpu.make_async_copy(f2w_ref.at[1], f2buf, sem.at[11])
    cp_f11.start()
    cp_f21.start()

    xn1 = _ln(x1_all, l1s_ref[1], l1b_ref[1])
    cps["k1"].wait()
    k1_all = _dot(xn1, a1[1])
    cps["v1"].wait()
    v1_all = _dot(xn1, a1[2])
    cps["q1"].wait()
    cps["o1"].wait()
    cp_f11.wait()
    cp_f21.wait()
    for b in range(B):
        xb = xn1[b * NQ:(b + 1) * NQ]
        qrow = _row_mm(xb[P:P + 1], a1[0])
        k = k1_all[b * NQ:(b + 1) * NQ]
        v = v1_all[b * NQ:(b + 1) * NQ]
        outs = []
        for h in range(NH):
            kh = k[:, h * D:(h + 1) * D]
            vh = v[:, h * D:(h + 1) * D]
            qh = qrow[:, h * D:(h + 1) * D]
            sc = jnp.sum(kh * qh, axis=1, keepdims=True) * (1.0 / 8.0)
            sc = sc - jnp.max(sc, axis=0, keepdims=True)
            w = jnp.exp(sc)
            pb = w / jnp.sum(w, axis=0, keepdims=True)
            outs.append(jnp.sum(pb * vh, axis=0, keepdims=True))
        attn = jnp.concatenate(outs, axis=-1)
        x2f = x2_all[b * NQ + P:b * NQ + P + 1] + _row_mm(attn, a1[3])
        hh = _ln(x2f, l2s_ref[1], l2b_ref[1])
        ffo1 = _row_mm(jax.nn.relu(_row_mm(hh, f1buf[...]) + f1b_ref[1]),
                       f2buf[...]) + f2b_ref[1]
        x1f = x1_all[b * NQ + P:b * NQ + P + 1] + ffo1
        hs = jnp.concatenate([x2f, x1f], axis=-1)
        hsn_ref[b:b + 1, :] = _ln(hs, lnfs_ref[...], lnfb_ref[...])


def _cone_call(g, pos, ln1_s, ln1_b, ln2_s, ln2_b, ff1_b, ff2_b,
               lnf_s, lnf_b, qw, kw, vw, ow, ff1_w, ff2_w):
    n_any = 6
    in_specs = ([pl.BlockSpec(memory_space=pl.ANY)
                 if i >= 10 else pl.BlockSpec(memory_space=pltpu.VMEM)
                 for i in range(10 + n_any)])
    return pl.pallas_call(
        _cone_body,
        in_specs=in_specs,
        out_specs=pl.BlockSpec(memory_space=pltpu.VMEM),
        out_shape=jax.ShapeDtypeStruct((B, 2 * H), jnp.float32),
        scratch_shapes=[
            pltpu.VMEM((4, H, H), jnp.float32),
            pltpu.VMEM((4, H, H), jnp.float32),
            pltpu.VMEM((H, FF), jnp.float32),
            pltpu.VMEM((FF, H), jnp.float32),
            pltpu.SemaphoreType.DMA((12,)),
        ],
    )(g, pos, ln1_s, ln1_b, ln2_s, ln2_b, ff1_b, ff2_b, lnf_s, lnf_b,
      qw, kw, vw, ow, ff1_w, ff2_w)


def _head_body(hsn_ref, g_ref, dw_ref, wih_ref, whh_ref, w_ref, o_ref,
               hout_sc):
    @pl.when(pl.program_id(0) == 0)
    def _():
        for b in range(B):
            hsn_b = hsn_ref[b:b + 1]
            ht = _row_mm(hsn_b, dw_ref[...])
            el = g_ref[(b + 1) * WIN - 1:(b + 1) * WIN]
            gates = (_row_mm(el, wih_ref[...]) +
                     _row_mm(ht, whh_ref[...]))
            gi = gates[:, 0 * H:1 * H]
            gf = gates[:, 1 * H:2 * H]
            gg = gates[:, 2 * H:3 * H]
            go = gates[:, 3 * H:4 * H]
            c = jax.nn.sigmoid(gf) * ht + jax.nn.sigmoid(gi) * jnp.tanh(gg)
            hout_sc[b:b + 1, :] = jax.nn.sigmoid(go) * jnp.tanh(c)

    w = w_ref[...]
    for b in range(B):
        hb = hout_sc[b:b + 1]
        col = jnp.sum(w * hb, axis=1, keepdims=True)
        o_ref[b:b + 1, :] = jnp.swapaxes(col, 0, 1)


def _head_call(hsn, g, dense_w, w_ih, w_hh, word_emb):
    return pl.pallas_call(
        _head_body,
        grid=(V // VT,),
        in_specs=[pl.BlockSpec((B, 2 * H), lambda j: (0, 0)),
                  pl.BlockSpec((NIDS, H), lambda j: (0, 0)),
                  pl.BlockSpec((2 * H, H), lambda j: (0, 0)),
                  pl.BlockSpec((H, 4 * H), lambda j: (0, 0)),
                  pl.BlockSpec((H, 4 * H), lambda j: (0, 0)),
                  pl.BlockSpec((VT, H), lambda j: (j, 0))],
        out_specs=pl.BlockSpec((B, VT), lambda j: (0, j)),
        out_shape=jax.ShapeDtypeStruct((B, V), jnp.float32),
        scratch_shapes=[pltpu.VMEM((B, H), jnp.float32)],
    )(hsn, g, dense_w, w_ih, w_hh, word_emb)


def kernel(input_ids, word_emb, pos_emb, qw, kw, vw, ow, ln1_s, ln1_b,
           ln2_s, ln2_b, ff1_w, ff1_b, ff2_w, ff2_b, lnf_s, lnf_b,
           dense_w, w_ih, w_hh):
    ids = input_ids[:, S - WIN:]
    ids_2d = ids.reshape(NIDS // GW, GW).astype(jnp.int32)
    pos = pos_emb[S - WIN:S]

    g = _gather_rows(word_emb, ids_2d)
    hsn = _cone_call(g, pos, ln1_s, ln1_b, ln2_s, ln2_b, ff1_b, ff2_b,
                     lnf_s, lnf_b, qw, kw, vw, ow, ff1_w, ff2_w)
    return _head_call(hsn, g, dense_w, w_ih, w_hh, word_emb)


# speedup vs baseline: 4.9000x; 1.0164x over previous
import functools

import jax
import jax.numpy as jnp
from jax.experimental import pallas as pl
from jax.experimental.pallas import tpu as pltpu
from jax.experimental.pallas import tpu_sc as plsc

V = 32000
H = 768
FF = 3072
NH = 12
B = 2
S = 2048
CHUNK = 64
D = H // NH
WIN = 3 * CHUNK
NQ = 2 * CHUNK
NIDS = B * WIN
GW = 24
VT = 3200
P = 126

_dot = functools.partial(jnp.dot, precision=jax.lax.Precision.HIGHEST,
                         preferred_element_type=jnp.float32)


def _dot_t(a, b):
    return jax.lax.dot_general(a, b, (((1,), (1,)), ((), ())),
                               precision=jax.lax.Precision.HIGHEST,
                               preferred_element_type=jnp.float32)


def _row_mm(x_row, w):
    return jnp.sum(x_row.reshape(-1, 1) * w, axis=0, keepdims=True)


def _ln(x, s, b, eps=1e-12):
    m = jnp.mean(x, axis=-1, keepdims=True)
    v = jnp.mean((x - m) ** 2, axis=-1, keepdims=True)
    return (x - m) / jnp.sqrt(v + eps) * s + b


def _gather_rows(table, ids_2d):
    mesh = plsc.VectorSubcoreMesh(core_axis_name="core",
                                  subcore_axis_name="subcore")

    @pl.kernel(out_type=jax.ShapeDtypeStruct((NIDS, H), jnp.float32),
               mesh=mesh)
    def kern(x_hbm, i_hbm, o_hbm):
        def body(i_vmem, o_vmem):
            pltpu.sync_copy(x_hbm.at[i_vmem.at[0]], o_vmem)

        pltpu.emit_pipeline(
            body,
            grid=(NIDS // GW,),
            in_specs=[pl.BlockSpec((1, GW), index_map=lambda i: (i, 0))],
            out_specs=[pl.BlockSpec((GW, H), index_map=lambda i: (i, 0))],
            core_axis_name="subcore",
            dimension_semantics=(pltpu.PARALLEL,),
        )(i_hbm, o_hbm)

    return kern(table, ids_2d)


def _cone_body(g_ref, pos_ref, l1s_ref, l1b_ref, l2s_ref, l2b_ref,
               f1b_ref, f2b_ref, lnfs_ref, lnfb_ref,
               qw_ref, kw_ref, vw_ref, ow_ref, f1w_ref, f2w_ref,
               hsn_ref,
               a0, a1, f1buf, f2buf, sem):
    cps = {
        "q0": pltpu.make_async_copy(qw_ref.at[0], a0.at[0], sem.at[0]),
        "k0": pltpu.make_async_copy(kw_ref.at[0], a0.at[1], sem.at[1]),
        "v0": pltpu.make_async_copy(vw_ref.at[0], a0.at[2], sem.at[2]),
        "o0": pltpu.make_async_copy(ow_ref.at[0], a0.at[3], sem.at[3]),
        "f10": pltpu.make_async_copy(f1w_ref.at[0], f1buf, sem.at[4]),
        "f20": pltpu.make_async_copy(f2w_ref.at[0], f2buf, sem.at[5]),
        "q1": pltpu.make_async_copy(qw_ref.at[1], a1.at[0], sem.at[6]),
        "k1": pltpu.make_async_copy(kw_ref.at[1], a1.at[1], sem.at[7]),
        "v1": pltpu.make_async_copy(vw_ref.at[1], a1.at[2], sem.at[8]),
        "o1": pltpu.make_async_copy(ow_ref.at[1], a1.at[3], sem.at[9]),
    }
    for c in cps.values():
        c.start()

    pos = pos_ref[...]
    g = g_ref[...]
    e_all = jnp.concatenate(
        [g[b * WIN:(b + 1) * WIN] + pos for b in range(B)], axis=0)
    xn = _ln(e_all, l1s_ref[0], l1b_ref[0])
    q_in = jnp.concatenate(
        [xn[b * WIN + CHUNK:(b + 1) * WIN] for b in range(B)], axis=0)

    cps["q0"].wait()
    q_all = _dot(q_in, a0[0])
    cps["k0"].wait()
    k_all = _dot(xn, a0[1])
    cps["v0"].wait()
    v_all = _dot(xn, a0[2])

    qi = jax.lax.broadcasted_iota(jnp.int32, (NQ, WIN), 0)
    kj = jax.lax.broadcasted_iota(jnp.int32, (NQ, WIN), 1)
    lo = (qi // CHUNK) * CHUNK
    mask = (kj >= lo) & (kj < lo + 2 * CHUNK)

    cps["o0"].wait()
    x2_rows = []
    for b in range(B):
        q = q_all[b * NQ:(b + 1) * NQ]
        k = k_all[b * WIN:(b + 1) * WIN]
        v = v_all[b * WIN:(b + 1) * WIN]
        outs = []
        for h in range(NH):
            sc = _dot_t(q[:, h * D:(h + 1) * D], k[:, h * D:(h + 1) * D])
            sc = sc * (1.0 / 8.0)
            sc = jnp.where(mask, sc, -1e9)
            p = jax.nn.softmax(sc, axis=-1)
            outs.append(_dot(p, v[:, h * D:(h + 1) * D]))
        attn = jnp.concatenate(outs, axis=-1)
        e_b = e_all[b * WIN + CHUNK:(b + 1) * WIN]
        x2_rows.append(e_b + _dot(attn, a0[3]))

    x2_all = jnp.concatenate(x2_rows, axis=0)
    h2 = _ln(x2_all, l2s_ref[0], l2b_ref[0])
    cps["f10"].wait()
    h1 = jax.nn.relu(_dot(h2, f1buf[...]) + f1b_ref[0])
    cps["f20"].wait()
    ffo = _dot(h1, f2buf[...]) + f2b_ref[0]
    x1_all = jnp.concatenate(
        [e_all[b * WIN + CHUNK:(b + 1) * WIN] for b in range(B)],
        axis=0) + ffo

    cp_f11 = pltpu.make_async_copy(f1w_ref.at[1], f1buf, sem.at[10])
    cp_f21 = pltpu.make_async_copy(f2w_ref.at[1], f2buf, sem.at[11])
    cp_f11.start()
    cp_f21.start()

    xn1 = _ln(x1_all, l1s_ref[1], l1b_ref[1])
    cps["k1"].wait()
    k1_all = _dot(xn1, a1[1])
    cps["v1"].wait()
    v1_all = _dot(xn1, a1[2])
    cps["q1"].wait()
    cps["o1"].wait()
    cp_f11.wait()
    cp_f21.wait()
    for b in range(B):
        xb = xn1[b * NQ:(b + 1) * NQ]
        qrow = _row_mm(xb[P:P + 1], a1[0])
        k = k1_all[b * NQ:(b + 1) * NQ]
        v = v1_all[b * NQ:(b + 1) * NQ]
        outs = []
        for h in range(NH):
            kh = k[:, h * D:(h + 1) * D]
            vh = v[:, h * D:(h + 1) * D]
            qh = qrow[:, h * D:(h + 1) * D]
            sc = jnp.sum(kh * qh, axis=1, keepdims=True) * (1.0 / 8.0)
            sc = sc - jnp.max(sc, axis=0, keepdims=True)
            w = jnp.exp(sc)
            pb = w / jnp.sum(w, axis=0, keepdims=True)
            outs.append(jnp.sum(pb * vh, axis=0, keepdims=True))
        attn = jnp.concatenate(outs, axis=-1)
        x2f = x2_all[b * NQ + P:b * NQ + P + 1] + _row_mm(attn, a1[3])
        hh = _ln(x2f, l2s_ref[1], l2b_ref[1])
        ffo1 = _row_mm(jax.nn.relu(_row_mm(hh, f1buf[...]) + f1b_ref[1]),
                       f2buf[...]) + f2b_ref[1]
        x1f = x1_all[b * NQ + P:b * NQ + P + 1] + ffo1
        hs = jnp.concatenate([x2f, x1f], axis=-1)
        hsn_ref[b:b + 1, :] = _ln(hs, lnfs_ref[...], lnfb_ref[...])


def _cone_call(g, pos, ln1_s, ln1_b, ln2_s, ln2_b, ff1_b, ff2_b,
               lnf_s, lnf_b, qw, kw, vw, ow, ff1_w, ff2_w):
    n_any = 6
    in_specs = ([pl.BlockSpec(memory_space=pl.ANY)
                 if i >= 10 else pl.BlockSpec(memory_space=pltpu.VMEM)
                 for i in range(10 + n_any)])
    return pl.pallas_call(
        _cone_body,
        in_specs=in_specs,
        out_specs=pl.BlockSpec(memory_space=pltpu.VMEM),
        out_shape=jax.ShapeDtypeStruct((B, 2 * H), jnp.float32),
        scratch_shapes=[
            pltpu.VMEM((4, H, H), jnp.float32),
            pltpu.VMEM((4, H, H), jnp.float32),
            pltpu.VMEM((H, FF), jnp.float32),
            pltpu.VMEM((FF, H), jnp.float32),
            pltpu.SemaphoreType.DMA((12,)),
        ],
    )(g, pos, ln1_s, ln1_b, ln2_s, ln2_b, ff1_b, ff2_b, lnf_s, lnf_b,
      qw, kw, vw, ow, ff1_w, ff2_w)


def _head_body(hsn_ref, g_ref, dw_ref, wih_ref, whh_ref, w_ref, o_ref,
               hout_sc):
    @pl.when(pl.program_id(0) == 0)
    def _():
        for b in range(B):
            hsn_b = hsn_ref[b:b + 1]
            ht = _row_mm(hsn_b, dw_ref[...])
            el = g_ref[(b + 1) * WIN - 1:(b + 1) * WIN]
            gates = (_row_mm(el, wih_ref[...]) +
                     _row_mm(ht, whh_ref[...]))
            gi = gates[:, 0 * H:1 * H]
            gf = gates[:, 1 * H:2 * H]
            gg = gates[:, 2 * H:3 * H]
            go = gates[:, 3 * H:4 * H]
            c = jax.nn.sigmoid(gf) * ht + jax.nn.sigmoid(gi) * jnp.tanh(gg)
            hout_sc[b:b + 1, :] = jax.nn.sigmoid(go) * jnp.tanh(c)

    w = w_ref[...]
    for b in range(B):
        hb = hout_sc[b:b + 1]
        col = jnp.sum(w * hb, axis=1, keepdims=True)
        o_ref[b:b + 1, :] = jnp.swapaxes(col, 0, 1)


def _head_call(hsn, g, dense_w, w_ih, w_hh, word_emb):
    return pl.pallas_call(
        _head_body,
        grid=(V // VT,),
        in_specs=[pl.BlockSpec((B, 2 * H), lambda j: (0, 0)),
                  pl.BlockSpec((NIDS, H), lambda j: (0, 0)),
                  pl.BlockSpec((2 * H, H), lambda j: (0, 0)),
                  pl.BlockSpec((H, 4 * H), lambda j: (0, 0)),
                  pl.BlockSpec((H, 4 * H), lambda j: (0, 0)),
                  pl.BlockSpec((VT, H), lambda j: (j, 0))],
        out_specs=pl.BlockSpec((B, VT), lambda j: (0, j)),
        out_shape=jax.ShapeDtypeStruct((B, V), jnp.float32),
        scratch_shapes=[pltpu.VMEM((B, H), jnp.float32)],
    )(hsn, g, dense_w, w_ih, w_hh, word_emb)


def kernel(input_ids, word_emb, pos_emb, qw, kw, vw, ow, ln1_s, ln1_b,
           ln2_s, ln2_b, ff1_w, ff1_b, ff2_w, ff2_b, lnf_s, lnf_b,
           dense_w, w_ih, w_hh):
    ids = input_ids[:, S - WIN:]
    ids_2d = ids.reshape(NIDS // GW, GW).astype(jnp.int32)
    pos = pos_emb[S - WIN:S]

    g = _gather_rows(word_emb, ids_2d)
    hsn = _cone_call(g, pos, ln1_s, ln1_b, ln2_s, ln2_b, ff1_b, ff2_b,
                     lnf_s, lnf_b, qw, kw, vw, ow, ff1_w, ff2_w)
    return _head_call(hsn, g, dense_w, w_ih, w_hh, word_emb)
